# Initial kernel scaffold; baseline (speedup 1.0000x reference)
import jax
import jax.numpy as jnp
from jax.experimental import pallas as pl


def kernel(buffer_img, buffer_label, x, y, idx):
    raise NotImplementedError("write your pallas kernel here")


# speedup vs baseline: 1.1507x; 1.1507x over previous
import jax
import jax.numpy as jnp
from jax.experimental import pallas as pl
from jax.experimental.pallas import tpu as pltpu

M = 50000
B = 1024
ROW = 3072
R = 400
G = M // R


def _body(sidx_ref, spos_ref, starts_ref, buf_ref, x_ref, lab_ref, y_ref,
          out_img_ref, out_lab_ref):
    g = pl.program_id(0)
    out_img_ref[...] = buf_ref[...]
    out_lab_ref[...] = lab_ref[...]
    start = starts_ref[g]
    end = starts_ref[g + 1]
    base = g * R

    def upd(j, carry):
        row = sidx_ref[j] - base
        src = spos_ref[j]
        out_img_ref[pl.ds(row, 1), :] = x_ref[pl.ds(src, 1), :]
        out_lab_ref[pl.ds(row, 1), :] = y_ref[pl.ds(src, 1), :]
        return carry

    jax.lax.fori_loop(start, end, upd, 0)


def _call(buf2, x2, lab2, y2, sidx, spos, starts, interpret=False):
    return pl.pallas_call(
        _body,
        grid=(G,),
        in_specs=[
            pl.BlockSpec(memory_space=pltpu.SMEM),
            pl.BlockSpec(memory_space=pltpu.SMEM),
            pl.BlockSpec(memory_space=pltpu.SMEM),
            pl.BlockSpec((R, ROW), lambda g: (g, 0)),
            pl.BlockSpec((B, ROW), lambda g: (0, 0)),
            pl.BlockSpec((R, 1), lambda g: (g, 0)),
            pl.BlockSpec((B, 1), lambda g: (0, 0)),
        ],
        out_specs=[
            pl.BlockSpec((R, ROW), lambda g: (g, 0)),
            pl.BlockSpec((R, 1), lambda g: (g, 0)),
        ],
        out_shape=[
            jax.ShapeDtypeStruct((M, ROW), jnp.float32),
            jax.ShapeDtypeStruct((M, 1), jnp.int32),
        ],
        interpret=interpret,
    )(sidx, spos, starts, buf2, x2, lab2, y2)


def kernel(buffer_img, buffer_label, x, y, idx):
    buf2 = buffer_img.reshape(M, ROW)
    x2 = x.reshape(B, ROW)
    lab2 = buffer_label.reshape(M, 1)
    y2 = y.reshape(B, 1)
    order = jnp.argsort(idx, stable=True).astype(jnp.int32)
    sidx = idx[order].astype(jnp.int32)
    edges = jnp.arange(0, M + 1, R, dtype=jnp.int32)
    starts = jnp.searchsorted(sidx, edges, side="left").astype(jnp.int32)
    out_img, out_lab = _call(buf2, x2, lab2, y2, sidx, order, starts)
    return out_img.reshape(buffer_img.shape), out_lab.reshape(buffer_label.shape)
